# speedup vs baseline: 1.7738x; 1.7738x over previous
import functools

import jax
import jax.numpy as jnp
from jax import lax
from jax.experimental import pallas as pl
from jax.experimental.pallas import tpu as pltpu
from jax.experimental.pallas import tpu_sc as plsc

_NUM_CORES = 2
_NUM_SUBCORES = 16
_NUM_WORKERS = _NUM_CORES * _NUM_SUBCORES

_CHUNK = 16


@functools.lru_cache(maxsize=None)
def _make_gather(n_total: int, vocab: int, d: int):
  n_per_w = n_total // _NUM_WORKERS
  chunks = n_per_w // _CHUNK
  assert chunks % 2 == 0 and chunks * _CHUNK == n_per_w

  mesh = plsc.VectorSubcoreMesh(core_axis_name="c", subcore_axis_name="s")

  @functools.partial(
      pl.kernel,
      out_type=jax.ShapeDtypeStruct((n_total, d), jnp.float32),
      mesh=mesh,
      scratch_types=[
          pltpu.VMEM((n_per_w,), jnp.int32),
          pltpu.VMEM((_CHUNK, d), jnp.float32),
          pltpu.VMEM((_CHUNK, d), jnp.float32),
          pltpu.SemaphoreType.DMA,
          pltpu.SemaphoreType.DMA,
      ],
  )
  def gather_kernel(ids_hbm, table_hbm, out_hbm, idx_v, rows0, rows1,
                    sem0, sem1):
    wid = lax.axis_index("s") * _NUM_CORES + lax.axis_index("c")
    base = wid * n_per_w
    pltpu.sync_copy(ids_hbm.at[pl.ds(base, n_per_w)], idx_v)

    def start_gather(g, buf, sem):
      off = pl.multiple_of(g * _CHUNK, 8)
      return pltpu.async_copy(
          table_hbm.at[idx_v.at[pl.ds(off, _CHUNK)]], buf, sem)

    def store(g, buf):
      row = pl.multiple_of(base + g * _CHUNK, 8)
      pltpu.sync_copy(buf, out_hbm.at[pl.ds(row, _CHUNK)])

    start_gather(0, rows0, sem0)

    bufs = (rows0, rows1)
    sems = (sem0, sem1)

    def body(i, carry):
      del carry
      for b in range(2):
        g = i * 2 + b
        nxt = g + 1

        @pl.when(nxt < chunks)
        def _():
          start_gather(nxt, bufs[(b + 1) % 2], sems[(b + 1) % 2])

        pltpu.make_async_copy(
            table_hbm.at[idx_v.at[pl.ds(0, _CHUNK)]], bufs[b], sems[b]
        ).wait()
        store(g, bufs[b])
      return 0

    lax.fori_loop(0, chunks // 2, body, 0, unroll=1)

  return gather_kernel


def kernel(input_ids, embed_tokens):
  b, s = input_ids.shape
  v, d = embed_tokens.shape
  n = b * s
  flat_ids = input_ids.reshape(n)
  out = _make_gather(n, v, d)(flat_ids, embed_tokens)
  return out.reshape(b, s, d)
